# Initial kernel scaffold; baseline (speedup 1.0000x reference)
import jax
import jax.numpy as jnp
from jax.experimental import pallas as pl


def kernel(x, edge_index, W):
    raise NotImplementedError("write your pallas kernel here")


# speedup vs baseline: 4.2665x; 4.2665x over previous
import functools

import jax
import jax.numpy as jnp
from jax import lax
from jax.experimental import pallas as pl
from jax.experimental.pallas import tpu as pltpu
from jax.experimental.pallas import tpu_sc as plsc

N_NODES = 10000
N_EDGES = 320000
IN_DIM = 128
OUT_DIM = 128

NC = 2
NS = 16
CHUNK = 128
CHUNKS_PER_TILE = 79
PAD_EDGES = NC * NS * CHUNKS_PER_TILE * CHUNK
ROWS_PER_TILE = 640
ACC_ROWS = NS * ROWS_PER_TILE


def _mm_body(x_ref, w_ref, o_ref):
    o_ref[...] = jnp.dot(x_ref[...], w_ref[...], preferred_element_type=jnp.float32)


def _matmul(x, w):
    grid = 10
    blk = N_NODES // grid
    return pl.pallas_call(
        _mm_body,
        grid=(grid,),
        in_specs=[
            pl.BlockSpec((blk, IN_DIM), lambda i: (i, 0)),
            pl.BlockSpec((IN_DIM, OUT_DIM), lambda i: (0, 0)),
        ],
        out_specs=pl.BlockSpec((blk, OUT_DIM), lambda i: (i, 0)),
        out_shape=jax.ShapeDtypeStruct((N_NODES, OUT_DIM), jnp.float32),
    )(x, w)


_sc_mesh = plsc.VectorSubcoreMesh(
    core_axis_name="c", subcore_axis_name="s", num_cores=NC, num_subcores=NS
)


@functools.partial(
    pl.kernel,
    out_type=jax.ShapeDtypeStruct((NC * ACC_ROWS, OUT_DIM), jnp.float32),
    mesh=_sc_mesh,
    scratch_types=[
        pltpu.VMEM((CHUNK,), jnp.int32),
        pltpu.VMEM((CHUNK,), jnp.int32),
        pltpu.VMEM((CHUNK, OUT_DIM), jnp.float32),
        pltpu.VMEM_SHARED((ACC_ROWS, OUT_DIM), jnp.float32),
        pltpu.SemaphoreType.DMA,
    ],
)
def _sc_aggregate(src_hbm, dst_hbm, h_hbm, z_hbm, out_hbm,
                  src_v, dst_v, rows_v, acc, sem):
    c = lax.axis_index("c")
    s = lax.axis_index("s")
    wid = c * NS + s

    pltpu.sync_copy(z_hbm, rows_v)
    for k in range(ROWS_PER_TILE // CHUNK):
        pltpu.sync_copy(
            rows_v, acc.at[pl.ds(s * ROWS_PER_TILE + k * CHUNK, CHUNK)]
        )
    plsc.subcore_barrier()

    base = wid * (CHUNKS_PER_TILE * CHUNK)

    def body(i, carry):
        e0 = base + i * CHUNK
        pltpu.sync_copy(src_hbm.at[pl.ds(e0, CHUNK)], src_v)
        pltpu.sync_copy(dst_hbm.at[pl.ds(e0, CHUNK)], dst_v)
        pltpu.async_copy(h_hbm.at[src_v], rows_v, sem).wait()
        pltpu.sync_copy(rows_v, acc.at[dst_v], add=True)
        return carry

    lax.fori_loop(0, CHUNKS_PER_TILE, body, 0)
    plsc.subcore_barrier()

    pltpu.sync_copy(
        acc.at[pl.ds(s * ROWS_PER_TILE, ROWS_PER_TILE)],
        out_hbm.at[pl.ds(c * ACC_ROWS + s * ROWS_PER_TILE, ROWS_PER_TILE)],
    )


def _combine_body(p_ref, o_ref):
    o_ref[...] = jnp.maximum(p_ref[0] + p_ref[1], 0.0)


def _combine(partials):
    grid = 10
    blk = N_NODES // grid
    return pl.pallas_call(
        _combine_body,
        grid=(grid,),
        in_specs=[pl.BlockSpec((NC, blk, OUT_DIM), lambda i: (0, i, 0))],
        out_specs=pl.BlockSpec((blk, OUT_DIM), lambda i: (i, 0)),
        out_shape=jax.ShapeDtypeStruct((N_NODES, OUT_DIM), jnp.float32),
    )(partials)


def kernel(x, edge_index, W):
    ei = edge_index.astype(jnp.int32)
    dst = ei[0]
    src = ei[1]
    pad = PAD_EDGES - N_EDGES
    src_p = jnp.concatenate([src, jnp.zeros((pad,), jnp.int32)])
    dst_p = jnp.concatenate([dst, jnp.full((pad,), N_NODES, jnp.int32)])
    zeros_rows = jnp.zeros((CHUNK, OUT_DIM), jnp.float32)

    h = _matmul(x, W)
    partials = _sc_aggregate(src_p, dst_p, h, zeros_rows)
    p2 = partials.reshape(NC, ACC_ROWS, OUT_DIM)[:, :N_NODES, :]
    return _combine(p2)
